# baseline (speedup 1.0000x reference)
import functools

import jax
import jax.numpy as jnp
from jax import lax
from jax.experimental import pallas as pl
from jax.experimental.pallas import tpu as pltpu
from jax.experimental.pallas import tpu_sc as plsc

_D = 64
_DP = 128
_B = 4096 * 200
_NC, _NS = 2, 16
_NW = _NC * _NS
_BPW = _B // _NW
_IW = 128
_K = 2
_CH = _K * _IW
_NCH = _BPW // _CH
_XR = _BPW // _IW

_mesh = plsc.VectorSubcoreMesh(core_axis_name="c", subcore_axis_name="s")

_TC = 512


def _transpose_pad_block(wt_ref, out_ref):
    blk = wt_ref[...]
    eye = jnp.eye(_D, dtype=jnp.float32)
    t = jax.lax.dot_general(
        blk, eye, (((0,), (0,)), ((), ())), preferred_element_type=jnp.float32
    )
    out_ref[:, 0:_D] = t
    out_ref[:, _D:_DP] = jnp.zeros((_TC, _DP - _D), jnp.float32)


def _transpose_pad(wt):
    grid = pl.cdiv(wt.shape[1], _TC)
    return pl.pallas_call(
        _transpose_pad_block,
        grid=(grid,),
        in_specs=[pl.BlockSpec((_D, _TC), lambda j: (0, j))],
        out_specs=pl.BlockSpec((_TC, _DP), lambda j: (j, 0)),
        out_shape=jax.ShapeDtypeStruct((wt.shape[1], _DP), jnp.float32),
    )(wt)


@functools.partial(
    pl.kernel,
    mesh=_mesh,
    out_type=jax.ShapeDtypeStruct((_B, _DP), jnp.float32),
    scratch_types=[
        pltpu.VMEM((_XR, _IW), jnp.int32),
        pltpu.VMEM((2, _CH, _DP), jnp.float32),
        pltpu.SemaphoreType.DMA,
        pltpu.SemaphoreType.DMA,
        pltpu.SemaphoreType.DMA,
        pltpu.SemaphoreType.DMA,
    ],
    compiler_params=pltpu.CompilerParams(use_tc_tiling_on_sc=True),
)
def _emb_lookup(x_hbm, w_hbm, out_hbm, idx_v, rows_v, g0, g1, o0, o1):
    wid = lax.axis_index("s") * _NC + lax.axis_index("c")
    base = wid * _BPW
    gsem = (g0, g1)
    osem = (o0, o1)

    pltpu.sync_copy(x_hbm.at[pl.ds(pl.multiple_of(wid * _XR, 8), _XR)], idx_v)

    def fire(c, b):
        for j in range(_K):
            pltpu.async_copy(
                w_hbm.at[idx_v.at[c * _K + j]],
                rows_v.at[b].at[pl.ds(j * _IW, _IW)],
                gsem[b],
            )

    def wait_gather(b):
        pltpu.make_async_copy(w_hbm.at[pl.ds(0, _CH)], rows_v.at[b], gsem[b]).wait()

    def start_store(c, b):
        off = pl.multiple_of(base + c * _CH, _CH)
        pltpu.async_copy(rows_v.at[b], out_hbm.at[pl.ds(off, _CH)], osem[b])

    def wait_store(b):
        pltpu.make_async_copy(rows_v.at[b], out_hbm.at[pl.ds(0, _CH)], osem[b]).wait()

    fire(0, 0)

    @pl.loop(0, _NCH, step=2)
    def _(g):
        for b in range(2):
            c = g + b

            @pl.when(c + 1 < _NCH)
            def _():
                @pl.when(c >= 1)
                def _():
                    wait_store(1 - b)

                fire(c + 1, 1 - b)

            wait_gather(b)
            start_store(c, b)

    wait_store(0)
    wait_store(1)


def kernel(x, w):
    xf = x.reshape(_B // _IW, _IW).astype(jnp.int32)
    wpad = _transpose_pad(w.T)
    out = _emb_lookup(xf, wpad)
    return out[:, : _D].reshape(x.shape[0], x.shape[1], _D)


# speedup vs baseline: 2.3340x; 2.3340x over previous
import functools

import jax
import jax.numpy as jnp
from jax import lax
from jax.experimental import pallas as pl
from jax.experimental.pallas import tpu as pltpu
from jax.experimental.pallas import tpu_sc as plsc

_D = 64
_DP = 128
_B = 4096 * 200
_NC, _NS = 2, 16
_NW = _NC * _NS
_BPW = _B // _NW
_IW = 128
_K = 2
_CH = _K * _IW
_NCH = _BPW // _CH
_XR = _BPW // _IW

_mesh = plsc.VectorSubcoreMesh(core_axis_name="c", subcore_axis_name="s")

_TC = 8192


def _transpose_pad_block(wt_ref, out_ref):
    blk = wt_ref[...]
    eye = jnp.eye(_D, dtype=jnp.float32)
    t = jax.lax.dot_general(
        blk, eye, (((0,), (0,)), ((), ())), preferred_element_type=jnp.float32
    )
    out_ref[:, 0:_D] = t
    out_ref[:, _D:_DP] = jnp.zeros((_TC, _DP - _D), jnp.float32)


def _transpose_pad(wt):
    grid = pl.cdiv(wt.shape[1], _TC)
    return pl.pallas_call(
        _transpose_pad_block,
        grid=(grid,),
        in_specs=[pl.BlockSpec((_D, _TC), lambda j: (0, j))],
        out_specs=pl.BlockSpec((_TC, _DP), lambda j: (j, 0)),
        out_shape=jax.ShapeDtypeStruct((wt.shape[1], _DP), jnp.float32),
    )(wt)


@functools.partial(
    pl.kernel,
    mesh=_mesh,
    out_type=jax.ShapeDtypeStruct((_B, _DP), jnp.float32),
    scratch_types=[
        pltpu.VMEM((_XR, _IW), jnp.int32),
        pltpu.VMEM((2, _CH, _DP), jnp.float32),
        pltpu.SemaphoreType.DMA,
        pltpu.SemaphoreType.DMA,
        pltpu.SemaphoreType.DMA,
        pltpu.SemaphoreType.DMA,
    ],
    compiler_params=pltpu.CompilerParams(use_tc_tiling_on_sc=True),
)
def _emb_lookup(x_hbm, w_hbm, out_hbm, idx_v, rows_v, g0, g1, o0, o1):
    wid = lax.axis_index("s") * _NC + lax.axis_index("c")
    base = wid * _BPW
    gsem = (g0, g1)
    osem = (o0, o1)

    pltpu.sync_copy(x_hbm.at[pl.ds(pl.multiple_of(wid * _XR, 8), _XR)], idx_v)

    def fire(c, b):
        for j in range(_K):
            pltpu.async_copy(
                w_hbm.at[idx_v.at[c * _K + j]],
                rows_v.at[b].at[pl.ds(j * _IW, _IW)],
                gsem[b],
            )

    def wait_gather(b):
        pltpu.make_async_copy(w_hbm.at[pl.ds(0, _CH)], rows_v.at[b], gsem[b]).wait()

    def start_store(c, b):
        off = pl.multiple_of(base + c * _CH, _CH)
        pltpu.async_copy(rows_v.at[b], out_hbm.at[pl.ds(off, _CH)], osem[b])

    def wait_store(b):
        pltpu.make_async_copy(rows_v.at[b], out_hbm.at[pl.ds(0, _CH)], osem[b]).wait()

    fire(0, 0)

    @pl.loop(0, _NCH, step=2)
    def _(g):
        for b in range(2):
            c = g + b

            @pl.when(c + 1 < _NCH)
            def _():
                @pl.when(c >= 1)
                def _():
                    wait_store(1 - b)

                fire(c + 1, 1 - b)

            wait_gather(b)
            start_store(c, b)

    wait_store(0)
    wait_store(1)


def kernel(x, w):
    xf = x.reshape(_B // _IW, _IW).astype(jnp.int32)
    wpad = _transpose_pad(w.T)
    out = _emb_lookup(xf, wpad)
    return out[:, : _D].reshape(x.shape[0], x.shape[1], _D)


# speedup vs baseline: 2.4035x; 1.0298x over previous
import functools

import jax
import jax.numpy as jnp
from jax import lax
from jax.experimental import pallas as pl
from jax.experimental.pallas import tpu as pltpu
from jax.experimental.pallas import tpu_sc as plsc

_D = 64
_DP = 128
_B = 4096 * 200
_NC, _NS = 2, 16
_NW = _NC * _NS
_BPW = _B // _NW
_IW = 128
_K = 2
_CH = _K * _IW
_NCH = _BPW // _CH
_XR = _BPW // _IW

_mesh = plsc.VectorSubcoreMesh(core_axis_name="c", subcore_axis_name="s")

_TC = 16384


def _transpose_pad_block(wt_ref, out_ref):
    blk = wt_ref[...]
    eye = jnp.eye(_D, dtype=jnp.float32)
    t = jax.lax.dot_general(
        blk, eye, (((0,), (0,)), ((), ())), preferred_element_type=jnp.float32
    )
    out_ref[:, 0:_D] = t
    out_ref[:, _D:_DP] = jnp.zeros((_TC, _DP - _D), jnp.float32)


def _transpose_pad(wt):
    grid = pl.cdiv(wt.shape[1], _TC)
    return pl.pallas_call(
        _transpose_pad_block,
        grid=(grid,),
        in_specs=[pl.BlockSpec((_D, _TC), lambda j: (0, j))],
        out_specs=pl.BlockSpec((_TC, _DP), lambda j: (j, 0)),
        out_shape=jax.ShapeDtypeStruct((wt.shape[1], _DP), jnp.float32),
    )(wt)


@functools.partial(
    pl.kernel,
    mesh=_mesh,
    out_type=jax.ShapeDtypeStruct((_B, _DP), jnp.float32),
    scratch_types=[
        pltpu.VMEM((_XR, _IW), jnp.int32),
        pltpu.VMEM((2, _CH, _DP), jnp.float32),
        pltpu.SemaphoreType.DMA,
        pltpu.SemaphoreType.DMA,
        pltpu.SemaphoreType.DMA,
        pltpu.SemaphoreType.DMA,
    ],
    compiler_params=pltpu.CompilerParams(use_tc_tiling_on_sc=True),
)
def _emb_lookup(x_hbm, w_hbm, out_hbm, idx_v, rows_v, g0, g1, o0, o1):
    wid = lax.axis_index("s") * _NC + lax.axis_index("c")
    base = wid * _BPW
    gsem = (g0, g1)
    osem = (o0, o1)

    pltpu.sync_copy(x_hbm.at[pl.ds(pl.multiple_of(wid * _XR, 8), _XR)], idx_v)

    def fire(c, b):
        for j in range(_K):
            pltpu.async_copy(
                w_hbm.at[idx_v.at[c * _K + j]],
                rows_v.at[b].at[pl.ds(j * _IW, _IW)],
                gsem[b],
            )

    def wait_gather(b):
        pltpu.make_async_copy(w_hbm.at[pl.ds(0, _CH)], rows_v.at[b], gsem[b]).wait()

    def start_store(c, b):
        off = pl.multiple_of(base + c * _CH, _CH)
        pltpu.async_copy(rows_v.at[b], out_hbm.at[pl.ds(off, _CH)], osem[b])

    def wait_store(b):
        pltpu.make_async_copy(rows_v.at[b], out_hbm.at[pl.ds(0, _CH)], osem[b]).wait()

    fire(0, 0)

    @pl.loop(0, _NCH, step=2)
    def _(g):
        for b in range(2):
            c = g + b

            @pl.when(c + 1 < _NCH)
            def _():
                @pl.when(c >= 1)
                def _():
                    wait_store(1 - b)

                fire(c + 1, 1 - b)

            wait_gather(b)
            start_store(c, b)

    wait_store(0)
    wait_store(1)


def kernel(x, w):
    xf = x.reshape(_B // _IW, _IW).astype(jnp.int32)
    wpad = _transpose_pad(w.T)
    out = _emb_lookup(xf, wpad)
    return out[:, : _D].reshape(x.shape[0], x.shape[1], _D)


# speedup vs baseline: 2.4319x; 1.0118x over previous
import functools

import jax
import jax.numpy as jnp
from jax import lax
from jax.experimental import pallas as pl
from jax.experimental.pallas import tpu as pltpu
from jax.experimental.pallas import tpu_sc as plsc

_D = 64
_DP = 128
_B = 4096 * 200
_NC, _NS = 2, 16
_NW = _NC * _NS
_BPW = _B // _NW
_IW = 128
_K = 2
_CH = _K * _IW
_NCH = _BPW // _CH
_XR = _BPW // _IW

_mesh = plsc.VectorSubcoreMesh(core_axis_name="c", subcore_axis_name="s")

_TC = 32768


def _transpose_pad_block(wt_ref, out_ref):
    blk = wt_ref[...]
    eye = jnp.eye(_D, dtype=jnp.float32)
    t = jax.lax.dot_general(
        blk, eye, (((0,), (0,)), ((), ())), preferred_element_type=jnp.float32
    )
    out_ref[:, 0:_D] = t
    out_ref[:, _D:_DP] = jnp.zeros((_TC, _DP - _D), jnp.float32)


def _transpose_pad(wt):
    grid = pl.cdiv(wt.shape[1], _TC)
    return pl.pallas_call(
        _transpose_pad_block,
        grid=(grid,),
        in_specs=[pl.BlockSpec((_D, _TC), lambda j: (0, j))],
        out_specs=pl.BlockSpec((_TC, _DP), lambda j: (j, 0)),
        out_shape=jax.ShapeDtypeStruct((wt.shape[1], _DP), jnp.float32),
    )(wt)


@functools.partial(
    pl.kernel,
    mesh=_mesh,
    out_type=jax.ShapeDtypeStruct((_B, _DP), jnp.float32),
    scratch_types=[
        pltpu.VMEM((_XR, _IW), jnp.int32),
        pltpu.VMEM((2, _CH, _DP), jnp.float32),
        pltpu.SemaphoreType.DMA,
        pltpu.SemaphoreType.DMA,
        pltpu.SemaphoreType.DMA,
        pltpu.SemaphoreType.DMA,
    ],
    compiler_params=pltpu.CompilerParams(use_tc_tiling_on_sc=True),
)
def _emb_lookup(x_hbm, w_hbm, out_hbm, idx_v, rows_v, g0, g1, o0, o1):
    wid = lax.axis_index("s") * _NC + lax.axis_index("c")
    base = wid * _BPW
    gsem = (g0, g1)
    osem = (o0, o1)

    pltpu.sync_copy(x_hbm.at[pl.ds(pl.multiple_of(wid * _XR, 8), _XR)], idx_v)

    def fire(c, b):
        for j in range(_K):
            pltpu.async_copy(
                w_hbm.at[idx_v.at[c * _K + j]],
                rows_v.at[b].at[pl.ds(j * _IW, _IW)],
                gsem[b],
            )

    def wait_gather(b):
        pltpu.make_async_copy(w_hbm.at[pl.ds(0, _CH)], rows_v.at[b], gsem[b]).wait()

    def start_store(c, b):
        off = pl.multiple_of(base + c * _CH, _CH)
        pltpu.async_copy(rows_v.at[b], out_hbm.at[pl.ds(off, _CH)], osem[b])

    def wait_store(b):
        pltpu.make_async_copy(rows_v.at[b], out_hbm.at[pl.ds(0, _CH)], osem[b]).wait()

    fire(0, 0)

    @pl.loop(0, _NCH, step=2)
    def _(g):
        for b in range(2):
            c = g + b

            @pl.when(c + 1 < _NCH)
            def _():
                @pl.when(c >= 1)
                def _():
                    wait_store(1 - b)

                fire(c + 1, 1 - b)

            wait_gather(b)
            start_store(c, b)

    wait_store(0)
    wait_store(1)


def kernel(x, w):
    xf = x.reshape(_B // _IW, _IW).astype(jnp.int32)
    wpad = _transpose_pad(w.T)
    out = _emb_lookup(xf, wpad)
    return out[:, : _D].reshape(x.shape[0], x.shape[1], _D)
